# baseline (speedup 1.0000x reference)
import numpy as np
import jax
import jax.numpy as jnp
from jax.experimental import pallas as pl
from jax.experimental.pallas import tpu as pltpu

N = 10000
NFEAT = 128
NHID = 16
NCLASS = 8

BM = 200
TK = 1024

_SLOPE = (1.0 / 8.0 + 1.0 / 3.0) / 2.0

NB = N // BM
NC = -(-N // TK)

_TILES = [(i, c) for i in range(NB) for c in range(NC)
          if min(TK * c + TK, N) > BM * (i + 1)]
_TILES.append((NB - 1, NC - 1))
_I_ARR = np.array([t[0] for t in _TILES], dtype=np.int32)
_C_ARR = np.array([t[1] for t in _TILES], dtype=np.int32)
_F_ARR = np.array(
    [1 if (k == 0 or _TILES[k][0] != _TILES[k - 1][0]) else 0
     for k in range(len(_TILES))], dtype=np.int32)
NT = len(_TILES)


def _small_mm_kernel(x_ref, w1_ref, w2_ref, w4_ref, b2_ref, b4_ref,
                     s1_ref, skip0_ref, skip1_ref):
    x = x_ref[...]
    s1_ref[...] = jnp.dot(x, w1_ref[...], preferred_element_type=jnp.float32)
    skip0_ref[...] = (
        jnp.dot(x, w2_ref[...], preferred_element_type=jnp.float32)
        + b2_ref[...])
    skip1_ref[...] = (
        jnp.dot(x, w4_ref[...], preferred_element_type=jnp.float32)
        + b4_ref[...])


def _pass1_kernel(a_ref, s1_ref, skip0_ref, b1_ref, w3_ref,
                  skip1_ref, b3_ref, s2_ref, part_ref, s2_s):
    i = pl.program_id(0)

    @pl.when(i == 0)
    def _():
        s2_s[...] = jnp.zeros((N, NCLASS), jnp.float32)

    a = a_ref[...]
    h = jnp.dot(a, s1_ref[...], preferred_element_type=jnp.float32)
    h = h + b1_ref[...] + skip0_ref[...]
    h = jnp.where(h >= 0, h, _SLOPE * h)
    s2_blk = jnp.dot(h, w3_ref[...], preferred_element_type=jnp.float32)
    s2_s[pl.ds(i * BM, BM), :] = s2_blk
    s2_ref[...] = s2_blk
    part_ref[...] = (
        jnp.dot(a, s2_s[...], preferred_element_type=jnp.float32)
        + b3_ref[...] + skip1_ref[...])


def _pass2_kernel(iidx_ref, cidx_ref, first_ref,
                  a_ref, s2_ref, part_ref, out_ref):
    t = pl.program_id(0)
    i = iidx_ref[t]
    cstart = cidx_ref[t] * TK
    thresh = (i + 1) * BM
    gcol = jax.lax.broadcasted_iota(jnp.int32, (BM, TK), 1) + cstart
    am = jnp.where((gcol >= thresh) & (gcol < N), a_ref[...], 0.0)
    grow = jax.lax.broadcasted_iota(jnp.int32, (TK, NCLASS), 0) + cstart
    s2m = jnp.where(grow < N, s2_ref[...], 0.0)
    contrib = jnp.dot(am, s2m, preferred_element_type=jnp.float32)

    @pl.when(first_ref[t] == 1)
    def _():
        out_ref[...] = part_ref[...] + contrib

    @pl.when(first_ref[t] == 0)
    def _():
        out_ref[...] = out_ref[...] + contrib


def kernel(x, adj, W1, b1, W2, b2, W3, b3, W4, b4):
    b1r = b1.reshape(1, NHID)
    b2r = b2.reshape(1, NHID)
    b3r = b3.reshape(1, NCLASS)
    b4r = b4.reshape(1, NCLASS)

    s1, skip0, skip1 = pl.pallas_call(
        _small_mm_kernel,
        out_shape=(
            jax.ShapeDtypeStruct((N, NHID), jnp.float32),
            jax.ShapeDtypeStruct((N, NHID), jnp.float32),
            jax.ShapeDtypeStruct((N, NCLASS), jnp.float32),
        ),
    )(x, W1, W2, W4, b2r, b4r)

    s2, part = pl.pallas_call(
        _pass1_kernel,
        grid=(NB,),
        in_specs=[
            pl.BlockSpec((BM, N), lambda i: (i, 0)),
            pl.BlockSpec((N, NHID), lambda i: (0, 0)),
            pl.BlockSpec((BM, NHID), lambda i: (i, 0)),
            pl.BlockSpec((1, NHID), lambda i: (0, 0)),
            pl.BlockSpec((NHID, NCLASS), lambda i: (0, 0)),
            pl.BlockSpec((BM, NCLASS), lambda i: (i, 0)),
            pl.BlockSpec((1, NCLASS), lambda i: (0, 0)),
        ],
        out_specs=(
            pl.BlockSpec((BM, NCLASS), lambda i: (i, 0)),
            pl.BlockSpec((BM, NCLASS), lambda i: (i, 0)),
        ),
        out_shape=(
            jax.ShapeDtypeStruct((N, NCLASS), jnp.float32),
            jax.ShapeDtypeStruct((N, NCLASS), jnp.float32),
        ),
        scratch_shapes=[pltpu.VMEM((N, NCLASS), jnp.float32)],
    )(adj, s1, skip0, b1r, W3, skip1, b3r)

    iarr = jnp.asarray(_I_ARR)
    carr = jnp.asarray(_C_ARR)
    farr = jnp.asarray(_F_ARR)

    out = pl.pallas_call(
        _pass2_kernel,
        grid_spec=pltpu.PrefetchScalarGridSpec(
            num_scalar_prefetch=3,
            grid=(NT,),
            in_specs=[
                pl.BlockSpec((BM, TK), lambda t, ia, ca, fa: (ia[t], ca[t])),
                pl.BlockSpec((TK, NCLASS), lambda t, ia, ca, fa: (ca[t], 0)),
                pl.BlockSpec((BM, NCLASS), lambda t, ia, ca, fa: (ia[t], 0)),
            ],
            out_specs=pl.BlockSpec(
                (BM, NCLASS), lambda t, ia, ca, fa: (ia[t], 0)),
        ),
        out_shape=jax.ShapeDtypeStruct((N, NCLASS), jnp.float32),
    )(iarr, carr, farr, adj, s2, part)

    return (out, W1, W2, W3, W4)


# speedup vs baseline: 1.5965x; 1.5965x over previous
import functools

import jax
import jax.numpy as jnp
from jax.experimental import pallas as pl
from jax.experimental.pallas import tpu as pltpu

N = 10000
NFEAT = 128
NHID = 16
NCLASS = 8

BM = 200
NB = N // BM
GROUP = 10
NG = NB // GROUP

_SLOPE = (1.0 / 8.0 + 1.0 / 3.0) / 2.0
_NS = NHID + NCLASS


def _round128(v):
    return min(-(-v // 128) * 128, N)


_WIDTHS = [_round128(BM * (GROUP * g + GROUP)) for g in range(NG)]


def _small_mm_kernel(x_ref, w1_ref, w2_ref, w4_ref, b2_ref, b4_ref,
                     s1_ref, skip0_ref, skip1_ref):
    x = x_ref[...]
    s1_ref[...] = jnp.dot(x, w1_ref[...], preferred_element_type=jnp.float32)
    skip0_ref[...] = (
        jnp.dot(x, w2_ref[...], preferred_element_type=jnp.float32)
        + b2_ref[...])
    skip1_ref[...] = (
        jnp.dot(x, w4_ref[...], preferred_element_type=jnp.float32)
        + b4_ref[...])


def _pass1_kernel(a_ref, s1_ref, skip0_ref, b1_ref, w3_ref,
                  skip1_ref, b3_ref, s2_ref, part_ref, s_s):
    i = pl.program_id(0)
    iblk = NB - 1 - i

    @pl.when(i == 0)
    def _():
        s_s[:, 0:NHID] = s1_ref[...]
        s_s[:, NHID:_NS] = jnp.zeros((N, NCLASS), jnp.float32)

    a = a_ref[...]
    r = jnp.dot(a, s_s[...], preferred_element_type=jnp.float32)
    h = r[:, 0:NHID] + b1_ref[...] + skip0_ref[...]
    h = jnp.where(h >= 0, h, _SLOPE * h)
    s2_blk = jnp.dot(h, w3_ref[...], preferred_element_type=jnp.float32)
    s_s[pl.ds(iblk * BM, BM), NHID:_NS] = s2_blk
    s2_ref[...] = s2_blk
    part_ref[...] = r[:, NHID:_NS] + b3_ref[...] + skip1_ref[...]


def _pass2_kernel(a_ref, s2_ref, part_ref, out_ref, *, g, w):
    i = pl.program_id(0)
    iblk = GROUP * g + i
    thresh = (iblk + 1) * BM
    row = jax.lax.broadcasted_iota(jnp.int32, (w, NCLASS), 0)
    s2m = jnp.where(row < thresh, s2_ref[...], 0.0)
    out_ref[...] = part_ref[...] + jnp.dot(
        a_ref[...], s2m, preferred_element_type=jnp.float32)


def kernel(x, adj, W1, b1, W2, b2, W3, b3, W4, b4):
    b1r = b1.reshape(1, NHID)
    b2r = b2.reshape(1, NHID)
    b3r = b3.reshape(1, NCLASS)
    b4r = b4.reshape(1, NCLASS)

    s1, skip0, skip1 = pl.pallas_call(
        _small_mm_kernel,
        out_shape=(
            jax.ShapeDtypeStruct((N, NHID), jnp.float32),
            jax.ShapeDtypeStruct((N, NHID), jnp.float32),
            jax.ShapeDtypeStruct((N, NCLASS), jnp.float32),
        ),
    )(x, W1, W2, W4, b2r, b4r)

    rev = lambda i: (NB - 1 - i, 0)
    s2, part = pl.pallas_call(
        _pass1_kernel,
        grid=(NB,),
        in_specs=[
            pl.BlockSpec((BM, N), rev),
            pl.BlockSpec((N, NHID), lambda i: (0, 0)),
            pl.BlockSpec((BM, NHID), rev),
            pl.BlockSpec((1, NHID), lambda i: (0, 0)),
            pl.BlockSpec((NHID, NCLASS), lambda i: (0, 0)),
            pl.BlockSpec((BM, NCLASS), rev),
            pl.BlockSpec((1, NCLASS), lambda i: (0, 0)),
        ],
        out_specs=(
            pl.BlockSpec((BM, NCLASS), rev),
            pl.BlockSpec((BM, NCLASS), rev),
        ),
        out_shape=(
            jax.ShapeDtypeStruct((N, NCLASS), jnp.float32),
            jax.ShapeDtypeStruct((N, NCLASS), jnp.float32),
        ),
        scratch_shapes=[pltpu.VMEM((N, _NS), jnp.float32)],
    )(adj, s1, skip0, b1r, W3, skip1, b3r)

    outs = []
    for g in range(NG):
        w = _WIDTHS[g]
        outs.append(pl.pallas_call(
            functools.partial(_pass2_kernel, g=g, w=w),
            grid=(GROUP,),
            in_specs=[
                pl.BlockSpec((BM, w), lambda i, g=g: (GROUP * g + i, 0)),
                pl.BlockSpec((w, NCLASS), lambda i: (0, 0)),
                pl.BlockSpec((BM, NCLASS), lambda i, g=g: (GROUP * g + i, 0)),
            ],
            out_specs=pl.BlockSpec((BM, NCLASS), lambda i: (i, 0)),
            out_shape=jax.ShapeDtypeStruct((GROUP * BM, NCLASS), jnp.float32),
        )(adj, s2, part))

    out = jnp.concatenate(outs, axis=0)
    return (out, W1, W2, W3, W4)
